# baseline (speedup 1.0000x reference)
import functools

import jax
import jax.numpy as jnp
import numpy as np
from jax import lax
from jax.experimental import pallas as pl
from jax.experimental.pallas import tpu as pltpu
from jax.experimental.pallas import tpu_sc as plsc

_NW = 32
_LANES = 16
_IDX_PER_GATHER = 128
_STREAMS_PER_CHUNK = 1
_TABLE_LANES = 32


def _channel_perm(c):
    p = np.empty((c,), np.int32)
    for g in range(c // 32):
        for i in range(16):
            p[g * 32 + 2 * i] = g * 32 + i
            p[g * 32 + 2 * i + 1] = g * 32 + 16 + i
    return p


def _tc_node_transform(x2, w1, b1):
    c, n = x2.shape

    def body(x_ref, w_ref, b_ref, h_ref):
        h = lax.dot_general(
            x_ref[...], w_ref[...], (((0,), (1,)), ((), ())),
            preferred_element_type=jnp.float32)
        h_ref[...] = jnp.maximum(h + b_ref[...][None, :], 0.0).astype(h_ref.dtype)

    return pl.pallas_call(
        body,
        out_shape=jax.ShapeDtypeStruct((n, w1.shape[0]), jnp.bfloat16),
    )(x2, w1, b1)


def _tc_output_transform(x2, agg_pad, w2, b2, n):
    c = x2.shape[0]

    def body(x_ref, agg_ref, w2x_ref, w2a_ref, b_ref, o_ref):
        t1 = lax.dot_general(
            w2x_ref[...], x_ref[...], (((1,), (0,)), ((), ())),
            preferred_element_type=jnp.float32)
        agg = agg_ref[...][:n]
        t2 = lax.dot_general(
            w2a_ref[...], agg, (((1,), (1,)), ((), ())),
            preferred_element_type=jnp.float32)
        o_ref[...] = jnp.maximum(t1 + t2 + b_ref[...][:, None], 0.0)

    return pl.pallas_call(
        body,
        out_shape=jax.ShapeDtypeStruct((w2.shape[0], n), jnp.float32),
    )(x2, agg_pad, w2[:, :c], w2[:, c:], b2)


def _sc_gather_max(h, idx_resh, npw, cpw, nodes_per_chunk):
    n, cw = h.shape
    c = cw * 2
    mesh = plsc.VectorSubcoreMesh(core_axis_name="c", subcore_axis_name="s")
    k = _IDX_PER_GATHER // nodes_per_chunk
    groups = c // _TABLE_LANES
    ns = _STREAMS_PER_CHUNK
    rows_per_stream = _IDX_PER_GATHER // ns

    @functools.partial(
        pl.kernel,
        out_type=jax.ShapeDtypeStruct((_NW * npw, c), jnp.float32),
        mesh=mesh,
        compiler_params=pltpu.CompilerParams(
            use_tc_tiling_on_sc=False, needs_layout_passes=False),
        scratch_types=[
            pltpu.VMEM((cpw, _IDX_PER_GATHER), jnp.int32),
            pltpu.VMEM((2, _IDX_PER_GATHER, cw), jnp.int32),
            pltpu.VMEM((npw, c), jnp.float32),
            pltpu.VMEM_SHARED((n, cw), jnp.int32),
            [pltpu.SemaphoreType.DMA] * (2 * ns),
        ],
    )
    def run(h_hbm, idx_hbm, out_hbm, idx_v, rows_v, out_v, tab_v, sems):
        wid = lax.axis_index("s") * 2 + lax.axis_index("c")
        sid = lax.axis_index("s")
        rstep = n // 16
        pltpu.sync_copy(h_hbm.at[pl.ds(sid * rstep, rstep)],
                        tab_v.at[pl.ds(sid * rstep, rstep)])
        pltpu.sync_copy(idx_hbm.at[wid], idx_v)
        plsc.subcore_barrier()

        def start(chunk, buf):
            for s in range(ns):
                sl = pl.ds(s * rows_per_stream, rows_per_stream)
                pltpu.async_copy(
                    tab_v.at[idx_v.at[chunk, sl]], rows_v.at[buf, sl],
                    sems[buf * ns + s])

        def wait(buf):
            for s in range(ns):
                sl = pl.ds(s * rows_per_stream, rows_per_stream)
                pltpu.make_async_copy(
                    tab_v.at[idx_v.at[0, sl]], rows_v.at[buf, sl],
                    sems[buf * ns + s]).wait()

        def compute(chunk, buf):
            for j in range(nodes_per_chunk):
                node = chunk * nodes_per_chunk + j
                for g in range(groups):
                    sl = pl.ds(g * 16, 16)
                    acc = plsc.bitcast(rows_v[buf, j * k, sl], jnp.bfloat16)
                    for kk in range(1, k):
                        acc = jnp.maximum(acc, plsc.bitcast(
                            rows_v[buf, j * k + kk, sl], jnp.bfloat16))
                    lo, hi = plsc.unpack(acc, format=plsc.PackFormat.INTERLEAVED)
                    out_v[node, pl.ds(g * _TABLE_LANES, 16)] = lo
                    out_v[node, pl.ds(g * _TABLE_LANES + 16, 16)] = hi

        start(0, 0)
        wait(0)

        def pair_body(p, _):
            start(2 * p + 1, 1)
            compute(2 * p, 0)

            @pl.when(p < cpw // 2 - 1)
            def _():
                start(2 * p + 2, 0)

            wait(1)
            compute(2 * p + 1, 1)

            @pl.when(p < cpw // 2 - 1)
            def _():
                wait(0)
            return 0

        lax.fori_loop(0, cpw // 2, pair_body, 0)
        pltpu.sync_copy(out_v, out_hbm.at[pl.ds(wid * npw, npw)])

    return run(h, idx_resh)


def kernel(x, w1, b1, w2, b2, edge_index):
    b, c, n = x.shape[0], x.shape[1], x.shape[2]
    k = edge_index.shape[-1]
    c_out = w2.shape[0]

    x2 = x.reshape(c, n)
    idx_flat = edge_index[0].reshape(-1)

    nodes_per_chunk = _IDX_PER_GATHER // k
    npw = -(-n // _NW)
    chunk_nodes = nodes_per_chunk * 2
    npw = -(-npw // chunk_nodes) * chunk_nodes
    n_pad = _NW * npw
    cpw = (npw * k) // _IDX_PER_GATHER
    idx_pad = jnp.concatenate(
        [idx_flat, jnp.zeros((n_pad * k - n * k,), jnp.int32)])
    idx_resh = idx_pad.reshape(_NW, cpw, _IDX_PER_GATHER)

    perm = _channel_perm(c)
    h = _tc_node_transform(x2, w1[perm], b1[perm])
    h = jax.lax.bitcast_convert_type(
        h.reshape(n, c // 2, 2), jnp.int32)
    agg_pad = _sc_gather_max(h, idx_resh, npw, cpw, nodes_per_chunk)
    out2 = _tc_output_transform(x2, agg_pad, w2, b2, n)
    return out2.reshape(b, c_out, n, 1, 1)


# speedup vs baseline: 1.2592x; 1.2592x over previous
import functools

import jax
import jax.numpy as jnp
import numpy as np
from jax import lax
from jax.experimental import pallas as pl
from jax.experimental.pallas import tpu as pltpu
from jax.experimental.pallas import tpu_sc as plsc

_NW = 32
_IDX_PER_GATHER = 128
_TABLE_LANES = 32


def _half_perms(c):
    lo = np.empty((c // 2,), np.int32)
    hi = np.empty((c // 2,), np.int32)
    for g in range(c // 32):
        for i in range(16):
            lo[g * 16 + i] = g * 32 + i
            hi[g * 16 + i] = g * 32 + 16 + i
    return lo, hi


def _tc_node_transform_packed(x2, w1lo, w1hi, b1lo, b1hi):
    c, n = x2.shape

    def body(x_ref, wl_ref, wh_ref, bl_ref, bh_ref, h_ref):
        xb = x_ref[...]
        hl = lax.dot_general(
            xb, wl_ref[...], (((0,), (1,)), ((), ())),
            preferred_element_type=jnp.float32)
        hh = lax.dot_general(
            xb, wh_ref[...], (((0,), (1,)), ((), ())),
            preferred_element_type=jnp.float32)
        hl = jnp.maximum(hl + bl_ref[...][None, :], 0.0).astype(jnp.bfloat16)
        hh = jnp.maximum(hh + bh_ref[...][None, :], 0.0).astype(jnp.bfloat16)
        ul = lax.bitcast_convert_type(hl, jnp.uint16).astype(jnp.uint32)
        uh = lax.bitcast_convert_type(hh, jnp.uint16).astype(jnp.uint32)
        h_ref[...] = lax.bitcast_convert_type((uh << 16) | ul, jnp.int32)

    return pl.pallas_call(
        body,
        out_shape=jax.ShapeDtypeStruct((n, c // 2), jnp.int32),
    )(x2, w1lo, w1hi, b1lo, b1hi)


def _tc_output_transform(x2, agg_pad, w2, b2, n):
    c = x2.shape[0]

    def body(x_ref, agg_ref, w2x_ref, w2a_ref, b_ref, o_ref):
        t1 = lax.dot_general(
            w2x_ref[...], x_ref[...], (((1,), (0,)), ((), ())),
            preferred_element_type=jnp.float32)
        agg = agg_ref[...][:n]
        t2 = lax.dot_general(
            w2a_ref[...], agg, (((1,), (1,)), ((), ())),
            preferred_element_type=jnp.float32)
        o_ref[...] = jnp.maximum(t1 + t2 + b_ref[...][:, None], 0.0)

    return pl.pallas_call(
        body,
        out_shape=jax.ShapeDtypeStruct((w2.shape[0], n), jnp.float32),
    )(x2, agg_pad, w2[:, :c], w2[:, c:], b2)


def _sc_gather_max(h, idx1d, npw, cpw, nodes_per_chunk):
    n, cw = h.shape
    c = cw * 2
    mesh = plsc.VectorSubcoreMesh(core_axis_name="c", subcore_axis_name="s")
    k = _IDX_PER_GATHER // nodes_per_chunk
    groups = c // _TABLE_LANES
    ns = _STREAMS_PER_CHUNK
    rows_per_stream = _IDX_PER_GATHER // ns
    total_idx = idx1d.shape[0]
    full_len = npw * k
    tail_len = total_idx - (_NW - 1) * full_len
    tail_pairs = tail_len // (2 * _IDX_PER_GATHER)

    @functools.partial(
        pl.kernel,
        out_type=jax.ShapeDtypeStruct((_NW * npw, c), jnp.float32),
        mesh=mesh,
        compiler_params=pltpu.CompilerParams(
            use_tc_tiling_on_sc=False, needs_layout_passes=False),
        scratch_types=[
            pltpu.VMEM((cpw * _IDX_PER_GATHER,), jnp.int32),
            pltpu.VMEM((2, _IDX_PER_GATHER, cw), jnp.int32),
            pltpu.VMEM((npw, c), jnp.float32),
            pltpu.VMEM_SHARED((n, cw), jnp.int32),
            [pltpu.SemaphoreType.DMA] * (2 * ns),
        ],
    )
    def run(h_hbm, idx_hbm, out_hbm, idx_v, rows_v, out_v, tab_v, sems):
        wid = lax.axis_index("s") * 2 + lax.axis_index("c")
        sid = lax.axis_index("s")
        rstep = n // 16
        pltpu.sync_copy(h_hbm.at[pl.ds(sid * rstep, rstep)],
                        tab_v.at[pl.ds(sid * rstep, rstep)])

        @pl.when(wid < _NW - 1)
        def _():
            pltpu.sync_copy(idx_hbm.at[pl.ds(wid * full_len, full_len)],
                            idx_v.at[pl.ds(0, full_len)])

        @pl.when(wid == _NW - 1)
        def _():
            pltpu.sync_copy(
                idx_hbm.at[pl.ds((_NW - 1) * full_len, tail_len)],
                idx_v.at[pl.ds(0, tail_len)])

        npairs = lax.select(wid == _NW - 1, tail_pairs, cpw // 2)
        plsc.subcore_barrier()

        def start(chunk, buf):
            for s in range(ns):
                sl = pl.ds(chunk * _IDX_PER_GATHER + s * rows_per_stream,
                           rows_per_stream)
                dsl = pl.ds(s * rows_per_stream, rows_per_stream)
                pltpu.async_copy(
                    tab_v.at[idx_v.at[sl]], rows_v.at[buf, dsl],
                    sems[buf * ns + s])

        def wait(buf):
            for s in range(ns):
                dsl = pl.ds(s * rows_per_stream, rows_per_stream)
                pltpu.make_async_copy(
                    tab_v.at[idx_v.at[pl.ds(0, rows_per_stream)]],
                    rows_v.at[buf, dsl], sems[buf * ns + s]).wait()

        def compute(chunk, buf):
            for j in range(nodes_per_chunk):
                node = chunk * nodes_per_chunk + j
                for g in range(groups):
                    sl = pl.ds(g * 16, 16)
                    acc = plsc.bitcast(rows_v[buf, j * k, sl], jnp.bfloat16)
                    for kk in range(1, k):
                        acc = jnp.maximum(acc, plsc.bitcast(
                            rows_v[buf, j * k + kk, sl], jnp.bfloat16))
                    lo, hi = plsc.unpack(acc, format=plsc.PackFormat.INTERLEAVED)
                    out_v[node, pl.ds(g * _TABLE_LANES, 16)] = lo
                    out_v[node, pl.ds(g * _TABLE_LANES + 16, 16)] = hi

        start(0, 0)
        wait(0)

        def pair_body(p, _):
            start(2 * p + 1, 1)
            compute(2 * p, 0)

            @pl.when(p < npairs - 1)
            def _():
                start(2 * p + 2, 0)

            wait(1)
            compute(2 * p + 1, 1)

            @pl.when(p < npairs - 1)
            def _():
                wait(0)
            return 0

        lax.fori_loop(0, npairs, pair_body, 0)
        pltpu.sync_copy(out_v, out_hbm.at[pl.ds(wid * npw, npw)])

    return run(h, idx1d)


_STREAMS_PER_CHUNK = 2


def kernel(x, w1, b1, w2, b2, edge_index):
    b, c, n = x.shape[0], x.shape[1], x.shape[2]
    k = edge_index.shape[-1]
    c_out = w2.shape[0]

    x2 = x.reshape(c, n)
    idx1d = edge_index[0].reshape(-1)

    nodes_per_chunk = _IDX_PER_GATHER // k
    npw = -(-n // _NW)
    chunk_nodes = nodes_per_chunk * 2
    npw = -(-npw // chunk_nodes) * chunk_nodes
    cpw = (npw * k) // _IDX_PER_GATHER

    plo, phi = _half_perms(c)
    h = _tc_node_transform_packed(
        x2, w1[plo], w1[phi], b1[plo], b1[phi])
    agg_pad = _sc_gather_max(h, idx1d, npw, cpw, nodes_per_chunk)
    out2 = _tc_output_transform(x2, agg_pad, w2, b2, n)
    return out2.reshape(b, c_out, n, 1, 1)


# speedup vs baseline: 1.3094x; 1.0399x over previous
import functools

import jax
import jax.numpy as jnp
from jax import lax
from jax.experimental import pallas as pl
from jax.experimental.pallas import tpu as pltpu
from jax.experimental.pallas import tpu_sc as plsc

_NW = 32
_IDX_PER_GATHER = 128
_TABLE_LANES = 32


def _tc_node_transform_packed(x2, w1, b1):
    c, n = x2.shape

    def body(x_ref, w_ref, b_ref, h_ref):
        xb = x_ref[...]
        w = w_ref[...]
        bias = b_ref[...]
        hl = lax.dot_general(
            xb, w[: c // 2], (((0,), (1,)), ((), ())),
            preferred_element_type=jnp.float32)
        hh = lax.dot_general(
            xb, w[c // 2:], (((0,), (1,)), ((), ())),
            preferred_element_type=jnp.float32)
        hl = jnp.maximum(hl + bias[None, : c // 2], 0.0).astype(jnp.bfloat16)
        hh = jnp.maximum(hh + bias[None, c // 2:], 0.0).astype(jnp.bfloat16)
        ul = lax.bitcast_convert_type(hl, jnp.uint16).astype(jnp.uint32)
        uh = lax.bitcast_convert_type(hh, jnp.uint16).astype(jnp.uint32)
        h_ref[...] = lax.bitcast_convert_type((uh << 16) | ul, jnp.int32)

    return pl.pallas_call(
        body,
        out_shape=jax.ShapeDtypeStruct((n, c // 2), jnp.int32),
    )(x2, w1, b1)


def _tc_output_transform(x2, agg_pad, w2, b2, n):
    c = x2.shape[0]

    def body(x_ref, agg_ref, w2_ref, b_ref, o_ref):
        xb = x_ref[...]
        w2f = w2_ref[...]
        t1 = lax.dot_general(
            w2f[:, :c], xb, (((1,), (0,)), ((), ())),
            preferred_element_type=jnp.float32)
        agg = agg_ref[...][:n]
        t2 = lax.dot_general(
            w2f[:, c:], agg, (((1,), (1,)), ((), ())),
            preferred_element_type=jnp.float32)
        o_ref[...] = jnp.maximum(t1 + t2 + b_ref[...][:, None], 0.0)

    return pl.pallas_call(
        body,
        out_shape=jax.ShapeDtypeStruct((w2.shape[0], n), jnp.float32),
    )(x2, agg_pad, w2, b2)


def _sc_gather_max(h, idx1d, npw, cpw, nodes_per_chunk):
    n, cw = h.shape
    c = cw * 2
    mesh = plsc.VectorSubcoreMesh(core_axis_name="c", subcore_axis_name="s")
    k = _IDX_PER_GATHER // nodes_per_chunk
    groups = c // _TABLE_LANES
    ns = _STREAMS_PER_CHUNK
    rows_per_stream = _IDX_PER_GATHER // ns
    total_idx = idx1d.shape[0]
    full_len = npw * k
    tail_len = total_idx - (_NW - 1) * full_len
    tail_pairs = tail_len // (2 * _IDX_PER_GATHER)

    @functools.partial(
        pl.kernel,
        out_type=jax.ShapeDtypeStruct((_NW * npw, c), jnp.float32),
        mesh=mesh,
        compiler_params=pltpu.CompilerParams(
            use_tc_tiling_on_sc=False, needs_layout_passes=False),
        scratch_types=[
            pltpu.VMEM((cpw * _IDX_PER_GATHER,), jnp.int32),
            pltpu.VMEM((2, _IDX_PER_GATHER, cw), jnp.int32),
            pltpu.VMEM((npw, c), jnp.float32),
            pltpu.VMEM_SHARED((n, cw), jnp.int32),
            [pltpu.SemaphoreType.DMA] * (2 * ns),
        ],
    )
    def run(h_hbm, idx_hbm, out_hbm, idx_v, rows_v, out_v, tab_v, sems):
        wid = lax.axis_index("s") * 2 + lax.axis_index("c")
        sid = lax.axis_index("s")
        rstep = n // 16
        pltpu.sync_copy(h_hbm.at[pl.ds(sid * rstep, rstep)],
                        tab_v.at[pl.ds(sid * rstep, rstep)])

        @pl.when(wid < _NW - 1)
        def _():
            pltpu.sync_copy(idx_hbm.at[pl.ds(wid * full_len, full_len)],
                            idx_v.at[pl.ds(0, full_len)])

        @pl.when(wid == _NW - 1)
        def _():
            pltpu.sync_copy(
                idx_hbm.at[pl.ds((_NW - 1) * full_len, tail_len)],
                idx_v.at[pl.ds(0, tail_len)])

        npairs = lax.select(wid == _NW - 1, tail_pairs, cpw // 2)
        plsc.subcore_barrier()

        def start(chunk, buf):
            for s in range(ns):
                sl = pl.ds(chunk * _IDX_PER_GATHER + s * rows_per_stream,
                           rows_per_stream)
                dsl = pl.ds(s * rows_per_stream, rows_per_stream)
                pltpu.async_copy(
                    tab_v.at[idx_v.at[sl]], rows_v.at[buf, dsl],
                    sems[buf * ns + s])

        def wait(buf):
            for s in range(ns):
                dsl = pl.ds(s * rows_per_stream, rows_per_stream)
                pltpu.make_async_copy(
                    tab_v.at[idx_v.at[pl.ds(0, rows_per_stream)]],
                    rows_v.at[buf, dsl], sems[buf * ns + s]).wait()

        def compute(chunk, buf):
            for j in range(nodes_per_chunk):
                node = chunk * nodes_per_chunk + j
                for g in range(groups):
                    sl = pl.ds(g * 16, 16)
                    acc = plsc.bitcast(rows_v[buf, j * k, sl], jnp.bfloat16)
                    for kk in range(1, k):
                        acc = jnp.maximum(acc, plsc.bitcast(
                            rows_v[buf, j * k + kk, sl], jnp.bfloat16))
                    lo, hi = plsc.unpack(acc, format=plsc.PackFormat.INTERLEAVED)
                    out_v[node, pl.ds(g * 16, 16)] = lo
                    out_v[node, pl.ds(c // 2 + g * 16, 16)] = hi

        start(0, 0)
        wait(0)

        def pair_body(p, _):
            start(2 * p + 1, 1)
            compute(2 * p, 0)

            @pl.when(p < npairs - 1)
            def _():
                start(2 * p + 2, 0)

            wait(1)
            compute(2 * p + 1, 1)

            @pl.when(p < npairs - 1)
            def _():
                wait(0)
            return 0

        lax.fori_loop(0, npairs, pair_body, 0)
        pltpu.sync_copy(out_v, out_hbm.at[pl.ds(wid * npw, npw)])

    return run(h, idx1d)


_STREAMS_PER_CHUNK = 2


def kernel(x, w1, b1, w2, b2, edge_index):
    b, c, n = x.shape[0], x.shape[1], x.shape[2]
    k = edge_index.shape[-1]
    c_out = w2.shape[0]

    x2 = x.reshape(c, n)
    idx1d = edge_index[0].reshape(-1)

    nodes_per_chunk = _IDX_PER_GATHER // k
    npw = -(-n // _NW)
    chunk_nodes = nodes_per_chunk * 2
    npw = -(-npw // chunk_nodes) * chunk_nodes
    cpw = (npw * k) // _IDX_PER_GATHER

    h = _tc_node_transform_packed(x2, w1, b1)
    agg_pad = _sc_gather_max(h, idx1d, npw, cpw, nodes_per_chunk)
    out2 = _tc_output_transform(x2, agg_pad, w2, b2, n)
    return out2.reshape(b, c_out, n, 1, 1)


# speedup vs baseline: 1.3116x; 1.0016x over previous
import functools

import jax
import jax.numpy as jnp
from jax import lax
from jax.experimental import pallas as pl
from jax.experimental.pallas import tpu as pltpu
from jax.experimental.pallas import tpu_sc as plsc

_NW = 32
_IDX_PER_GATHER = 128
_TABLE_LANES = 32


def _tc_node_transform_packed(x2, w1, b1):
    c, n = x2.shape

    def body(x_ref, w_ref, b_ref, h_ref):
        xb = x_ref[...]
        w = w_ref[...]
        bias = b_ref[...]
        hl = lax.dot_general(
            xb, w[: c // 2], (((0,), (1,)), ((), ())),
            preferred_element_type=jnp.float32)
        hh = lax.dot_general(
            xb, w[c // 2:], (((0,), (1,)), ((), ())),
            preferred_element_type=jnp.float32)
        hl = jnp.maximum(hl + bias[None, : c // 2], 0.0).astype(jnp.bfloat16)
        hh = jnp.maximum(hh + bias[None, c // 2:], 0.0).astype(jnp.bfloat16)
        ul = lax.bitcast_convert_type(hl, jnp.uint16).astype(jnp.uint32)
        uh = lax.bitcast_convert_type(hh, jnp.uint16).astype(jnp.uint32)
        h_ref[...] = lax.bitcast_convert_type((uh << 16) | ul, jnp.int32)

    return pl.pallas_call(
        body,
        out_shape=jax.ShapeDtypeStruct((n, c // 2), jnp.int32),
    )(x2, w1, b1)


def _tc_output_transform(x2, agg_pad, w2, b2, n):
    c = x2.shape[0]

    def body(x_ref, agg_ref, w2_ref, b_ref, o_ref):
        xb = x_ref[...]
        w2f = w2_ref[...]
        t1 = lax.dot_general(
            w2f[:, :c], xb, (((1,), (0,)), ((), ())),
            preferred_element_type=jnp.float32)
        agg = agg_ref[...][:n]
        t2 = lax.dot_general(
            w2f[:, c:], agg, (((1,), (1,)), ((), ())),
            preferred_element_type=jnp.float32)
        o_ref[...] = jnp.maximum(t1 + t2 + b_ref[...][:, None], 0.0)

    return pl.pallas_call(
        body,
        out_shape=jax.ShapeDtypeStruct((w2.shape[0], n), jnp.float32),
    )(x2, agg_pad, w2, b2)


def _sc_gather_max(h, idx1d, npw, cpw, nodes_per_chunk):
    n, cw = h.shape
    c = cw * 2
    mesh = plsc.VectorSubcoreMesh(core_axis_name="c", subcore_axis_name="s")
    k = _IDX_PER_GATHER // nodes_per_chunk
    groups = c // _TABLE_LANES
    ns = _STREAMS_PER_CHUNK
    rows_per_stream = _IDX_PER_GATHER // ns
    total_idx = idx1d.shape[0]
    full_len = npw * k
    tail_len = total_idx - (_NW - 1) * full_len
    tail_pairs = tail_len // (2 * _IDX_PER_GATHER)

    @functools.partial(
        pl.kernel,
        out_type=jax.ShapeDtypeStruct((_NW * npw, c), jnp.float32),
        mesh=mesh,
        compiler_params=pltpu.CompilerParams(
            use_tc_tiling_on_sc=False, needs_layout_passes=False),
        scratch_types=[
            pltpu.VMEM((cpw * _IDX_PER_GATHER,), jnp.int32),
            pltpu.VMEM((2, _IDX_PER_GATHER, cw), jnp.int32),
            pltpu.VMEM((npw, c), jnp.float32),
            pltpu.VMEM_SHARED((n, cw), jnp.int32),
            [pltpu.SemaphoreType.DMA] * (2 * ns),
        ],
    )
    def run(h_hbm, idx_hbm, out_hbm, idx_v, rows_v, out_v, tab_v, sems):
        wid = lax.axis_index("s") * 2 + lax.axis_index("c")
        sid = lax.axis_index("s")
        rstep = n // 16
        pltpu.sync_copy(h_hbm.at[pl.ds(sid * rstep, rstep)],
                        tab_v.at[pl.ds(sid * rstep, rstep)])

        @pl.when(wid < _NW - 1)
        def _():
            pltpu.sync_copy(idx_hbm.at[pl.ds(wid * full_len, full_len)],
                            idx_v.at[pl.ds(0, full_len)])

        @pl.when(wid == _NW - 1)
        def _():
            pltpu.sync_copy(
                idx_hbm.at[pl.ds((_NW - 1) * full_len, tail_len)],
                idx_v.at[pl.ds(0, tail_len)])

        npairs = lax.select(wid == _NW - 1, tail_pairs, cpw // 2)
        plsc.subcore_barrier()

        def start(chunk, buf):
            for s in range(ns):
                sl = pl.ds(chunk * _IDX_PER_GATHER + s * rows_per_stream,
                           rows_per_stream)
                dsl = pl.ds(s * rows_per_stream, rows_per_stream)
                pltpu.async_copy(
                    tab_v.at[idx_v.at[sl]], rows_v.at[buf, dsl],
                    sems[buf * ns + s])

        def wait(buf):
            for s in range(ns):
                dsl = pl.ds(s * rows_per_stream, rows_per_stream)
                pltpu.make_async_copy(
                    tab_v.at[idx_v.at[pl.ds(0, rows_per_stream)]],
                    rows_v.at[buf, dsl], sems[buf * ns + s]).wait()

        def compute(chunk, buf):
            for j in range(nodes_per_chunk):
                node = chunk * nodes_per_chunk + j
                for g in range(groups):
                    sl = pl.ds(g * 16, 16)
                    acc = plsc.bitcast(rows_v[buf, j * k, sl], jnp.bfloat16)
                    for kk in range(1, k):
                        acc = jnp.maximum(acc, plsc.bitcast(
                            rows_v[buf, j * k + kk, sl], jnp.bfloat16))
                    lo, hi = plsc.unpack(acc, format=plsc.PackFormat.INTERLEAVED)
                    out_v[node, pl.ds(g * 16, 16)] = lo
                    out_v[node, pl.ds(c // 2 + g * 16, 16)] = hi

        start(0, 0)
        wait(0)

        def pair_body(p, _):
            start(2 * p + 1, 1)
            compute(2 * p, 0)

            @pl.when(p < npairs - 1)
            def _():
                start(2 * p + 2, 0)

            wait(1)
            compute(2 * p + 1, 1)

            @pl.when(p < npairs - 1)
            def _():
                wait(0)
            return 0

        lax.fori_loop(0, npairs, pair_body, 0)
        pltpu.sync_copy(out_v, out_hbm.at[pl.ds(wid * npw, npw)])

    return run(h, idx1d)


_STREAMS_PER_CHUNK = 2


def kernel(x, w1, b1, w2, b2, edge_index):
    b, c, n = x.shape[0], x.shape[1], x.shape[2]
    k = edge_index.shape[-1]
    c_out = w2.shape[0]

    x2 = jnp.squeeze(x, (0, 3, 4))
    idx1d = edge_index[0].reshape(-1)

    nodes_per_chunk = _IDX_PER_GATHER // k
    npw = -(-n // _NW)
    chunk_nodes = nodes_per_chunk * 2
    npw = -(-npw // chunk_nodes) * chunk_nodes
    cpw = (npw * k) // _IDX_PER_GATHER

    h = _tc_node_transform_packed(x2, w1, b1)
    agg_pad = _sc_gather_max(h, idx1d, npw, cpw, nodes_per_chunk)
    out2 = _tc_output_transform(x2, agg_pad, w2, b2, n)
    return out2.reshape(b, c_out, n, 1, 1)
